# baseline (speedup 1.0000x reference)
import functools

import jax
import jax.numpy as jnp
from jax import lax
from jax.experimental import pallas as pl
from jax.experimental.pallas import tpu as pltpu
from jax.experimental.pallas import tpu_sc as plsc

D = 32
B = 16384
L = 16
NW = 32
BPW = B // NW
NMC = 7813
NTC = 782
WC = 4
SEQCAP = 2048
KCAP = 2048
OCAP = 256
NREG = NW * SEQCAP
DUMP = B
ICH = 128


def _make_k1():
    mesh = plsc.VectorSubcoreMesh(core_axis_name="c", subcore_axis_name="s")

    @functools.partial(
        pl.kernel,
        mesh=mesh,
        out_type=(
            jax.ShapeDtypeStruct((NREG * D,), jnp.float32),
            jax.ShapeDtypeStruct((NREG * D,), jnp.float32),
            jax.ShapeDtypeStruct((NREG,), jnp.int32),
            jax.ShapeDtypeStruct((NREG,), jnp.int32),
        ),
        compiler_params=pltpu.CompilerParams(
            needs_layout_passes=False, use_tc_tiling_on_sc=True),
        scratch_types=[
            pltpu.VMEM((B,), jnp.int32),
            pltpu.VMEM((2, D, WC * 128), jnp.float32),
            pltpu.VMEM((2 * ICH * D,), jnp.float32),
            pltpu.VMEM((SEQCAP,), jnp.int32),
            pltpu.VMEM((B,), jnp.int32),
            pltpu.VMEM((B,), jnp.int32),
            pltpu.VMEM((256 * 16,), jnp.int32),
            pltpu.VMEM((OCAP,), jnp.int32),
            pltpu.VMEM((OCAP,), jnp.int32),
            pltpu.SMEM((260 + B // L,), jnp.int32),
            pltpu.SemaphoreType.DMA,
            pltpu.SemaphoreType.DMA,
        ],
    )
    def k1(model_hbm, task_hbm, mt_hbm, tt_hbm,
           mrows_hbm, trows_hbm, pmb_hbm, ptb_hbm,
           idxbuf, wins, stage, pstage, kb, kv, slots, ovb, ovv,
           counts, sem0, sem1):
        wid = lax.axis_index("s") * 2 + lax.axis_index("c")
        sid = lax.axis_index("s")
        region = wid * SEQCAP
        iota = lax.iota(jnp.int32, L)
        lane0 = iota == 0

        def phase(idx_hbm, tab_hbm, ncols, rows_hbm, pb_hbm):
            cs = (ncols * wid) // NW
            ce = (ncols * (wid + 1)) // NW
            nwin = (ce - cs + WC - 1) // WC

            def zcnt(i, c):
                counts[i] = 0
                return c
            lax.fori_loop(0, 260, zcnt, 0)

            dump16 = jnp.full((L,), DUMP, jnp.int32)

            def zperm(i, c):
                pstage[pl.ds(pl.multiple_of(i * L, L), L)] = dump16
                return c
            lax.fori_loop(0, SEQCAP // L, zperm, 0)

            pltpu.async_copy(idx_hbm, idxbuf, sem1).wait()

            def scan_body(i, c):
                off = pl.ds(pl.multiple_of(i * L, L), L)
                m = idxbuf[off]
                col = m >> 7
                mask = (col >= cs) & (col < ce)
                bvec = i * L + iota
                plsc.store_compressed(kv.at[off], m, mask=mask)
                plsc.store_compressed(kb.at[off], bvec, mask=mask)
                cnt = plsc.all_reduce_population_count(mask)
                counts[260 + i] = cnt[0]
                return c

            lax.fori_loop(0, B // L, scan_body, 0)

            def dyn_lane(ref, i):
                vec = ref[pl.ds(pl.multiple_of((i >> 4) * L, L), L)]
                lane = jnp.full((L,), i & 15, jnp.int32)
                return vec.at[lane].get(mode="promise_in_bounds")[0]

            def place_one(i, c):
                m = dyn_lane(kv, i)
                b = dyn_lane(kb, i)
                crel = (m >> 7) - cs
                old = counts[crel]
                counts[crel] = old + 1
                val = (b << 7) | (m & 127)

                @pl.when(old < 16)
                def _():
                    plsc.store_scatter(
                        slots,
                        [jnp.full((L,), crel * 16 + old, jnp.int32)],
                        jnp.full((L,), val, jnp.int32), mask=lane0)

                @pl.when(old >= 16)
                def _():
                    oo = counts[256]
                    counts[256] = oo + 1
                    oo = jnp.minimum(oo, OCAP - 1)
                    plsc.store_scatter(
                        ovb, [jnp.full((L,), oo, jnp.int32)],
                        jnp.full((L,), b, jnp.int32), mask=lane0)
                    plsc.store_scatter(
                        ovv, [jnp.full((L,), oo, jnp.int32)],
                        jnp.full((L,), m, jnp.int32), mask=lane0)
                return c

            def place_bin(i, c):
                nb = counts[260 + i]

                @pl.when(nb > 0)
                def _():
                    lax.fori_loop(i * L, i * L + nb, place_one, 0)
                return c

            lax.fori_loop(0, B // L, place_bin, 0)

            def wcol0(wi):
                return jnp.minimum(cs + WC * wi, ncols - WC)

            def fire(wi):
                c0 = wcol0(wi)
                pltpu.async_copy(
                    tab_hbm.at[:, pl.ds(c0 * 128, WC * 128)],
                    wins.at[wi & 1], sem0)

            def emit_row(v0, v1, b, ns):
                s0 = pl.multiple_of((ns & (2 * ICH - 1)) * D, D)
                stage[pl.ds(s0, L)] = v0
                stage[pl.ds(s0 + L, L)] = v1
                plsc.store_scatter(
                    pstage, [jnp.full((L,), ns, jnp.int32)],
                    jnp.full((L,), b, jnp.int32), mask=lane0)
                return ns + 1

            def flush_blk(fl, partial):
                src = stage.at[pl.ds(
                    pl.multiple_of((fl & 1) * ICH * D, ICH * D), ICH * D)]
                blk = pl.multiple_of((region + fl * ICH) * D, ICH * D)
                pltpu.async_copy(
                    src, rows_hbm.at[pl.ds(blk, ICH * D)], sem1).wait()

            def flush_check(ns, fl):
                @pl.when((ns >> 7) > fl)
                def _():
                    flush_blk(fl, False)
                return jnp.where((ns >> 7) > fl, fl + 1, fl)

            def process(wi, par, ns):
                c0 = wcol0(wi)
                for cc in range(WC):
                    crel = c0 - cs + cc
                    nhit = jnp.minimum(counts[crel], 16)

                    def hit_body(j, ns):
                        s = dyn_lane(slots, crel * 16 + j)
                        ok = ns < SEQCAP

                        @pl.when(ok)
                        def _():
                            cpos = jnp.full(
                                (L,), cc * 128 + (s & 127), jnp.int32)
                            v0 = plsc.load_gather(wins.at[par], [iota, cpos])
                            v1 = plsc.load_gather(
                                wins.at[par], [iota + L, cpos])
                            emit_row(v0, v1, s >> 7, ns)
                        return ns + jnp.where(ok, 1, 0)

                    ns = lax.fori_loop(0, nhit, hit_body, ns)
                return ns

            fire(0)

            def win_body(w, carry):
                ns, fl = carry

                @pl.when(w + 1 < nwin)
                def _():
                    fire(w + 1)
                pltpu.make_async_copy(
                    tab_hbm.at[:, pl.ds(0, WC * 128)],
                    wins.at[0], sem0).wait()
                ns = process(w, w & 1, ns)
                fl = flush_check(ns, fl)
                return ns, fl

            nsf, flf = lax.fori_loop(0, nwin, win_body, (0, 0))

            novf = jnp.minimum(counts[256], OCAP)

            def ovf_body(i, carry):
                ns, fl = carry
                m = dyn_lane(ovv, i)
                b = dyn_lane(ovb, i)
                ok = ns < SEQCAP

                @pl.when(ok)
                def _():
                    cb = pl.multiple_of((m >> 7) << 7, 128)
                    pltpu.async_copy(
                        tab_hbm.at[:, pl.ds(cb, 128)],
                        wins.at[0, :, pl.ds(0, 128)], sem1).wait()
                    cpos = jnp.full((L,), m & 127, jnp.int32)
                    v0 = plsc.load_gather(wins.at[0], [iota, cpos])
                    v1 = plsc.load_gather(wins.at[0], [iota + L, cpos])
                    emit_row(v0, v1, b, ns)
                ns = ns + jnp.where(ok, 1, 0)
                fl = flush_check(ns, fl)
                return ns, fl

            nsf, flf = lax.fori_loop(0, novf, ovf_body, (nsf, flf))

            @pl.when((nsf & 127) != 0)
            def _():
                flush_blk(nsf >> 7, True)

            pltpu.async_copy(
                pstage, pb_hbm.at[pl.ds(region, SEQCAP)], sem1).wait()

        phase(model_hbm, mt_hbm, NMC, mrows_hbm, pmb_hbm)
        phase(task_hbm, tt_hbm, NTC, trows_hbm, ptb_hbm)

    return k1


INVN = B + SEQCAP
SLC = INVN // (NW // 2)


def _make_k2a():
    mesh = plsc.VectorSubcoreMesh(core_axis_name="c", subcore_axis_name="s")

    @functools.partial(
        pl.kernel,
        mesh=mesh,
        out_type=(
            jax.ShapeDtypeStruct((2, INVN), jnp.int32),
            jax.ShapeDtypeStruct((2, INVN), jnp.int32),
        ),
        compiler_params=pltpu.CompilerParams(
            needs_layout_passes=False, use_tc_tiling_on_sc=False),
        scratch_types=[
            pltpu.VMEM((SEQCAP // ICH, ICH), jnp.int32),
            pltpu.VMEM((SEQCAP // ICH, ICH), jnp.int32),
            pltpu.VMEM((SLC,), jnp.int32),
            pltpu.VMEM_SHARED((INVN,), jnp.int32),
            pltpu.SemaphoreType.DMA,
        ],
    )
    def k2a(pmb_hbm, ptb_hbm, invm_hbm, invt_hbm, pb, sv, zb, sinv, sem):
        cid = lax.axis_index("c")
        sid = lax.axis_index("s")
        wid = sid * 2 + cid
        region = wid * SEQCAP
        iota = lax.iota(jnp.int32, L)
        zero16 = jnp.zeros((L,), jnp.int32)

        def zfill(i, c):
            zb[pl.ds(pl.multiple_of(i * L, L), L)] = zero16
            return c
        lax.fori_loop(0, SLC // L, zfill, 0)

        def build_seq(i, c):
            sv[i // (ICH // L),
               pl.ds(pl.multiple_of((i % (ICH // L)) * L, L), L)] = (
                   region + i * L + iota + 1)
            return c
        lax.fori_loop(0, SEQCAP // L, build_seq, 0)

        def invert(p_hbm, inv_hbm):
            pltpu.sync_copy(zb, sinv.at[pl.ds(sid * SLC, SLC)])
            plsc.subcore_barrier()
            for j in range(SEQCAP // ICH):
                pltpu.sync_copy(
                    p_hbm.at[pl.ds(region + j * ICH, ICH)], pb.at[j])

            def adj(i, c):
                jj = i // (ICH // L)
                off = pl.ds(pl.multiple_of((i % (ICH // L)) * L, L), L)
                v = pb[jj, off]
                v = jnp.where(v == DUMP, DUMP + wid * 64 + (iota + i) % 64, v)
                pb[jj, off] = v
                return c
            lax.fori_loop(0, SEQCAP // L, adj, 0)

            for j in range(SEQCAP // ICH):
                pltpu.sync_copy(sv.at[j], sinv.at[pb.at[j]])
            plsc.subcore_barrier()
            pltpu.sync_copy(
                sinv.at[pl.ds(sid * SLC, SLC)],
                inv_hbm.at[cid, pl.ds(sid * SLC, SLC)])
            plsc.subcore_barrier()

        invert(pmb_hbm, invm_hbm)
        invert(ptb_hbm, invt_hbm)

    return k2a


def _make_k2b():
    mesh = plsc.VectorSubcoreMesh(core_axis_name="c", subcore_axis_name="s")

    @functools.partial(
        pl.kernel,
        mesh=mesh,
        out_type=jax.ShapeDtypeStruct((B,), jnp.float32),
        compiler_params=pltpu.CompilerParams(
            needs_layout_passes=False, use_tc_tiling_on_sc=False),
        scratch_types=[
            pltpu.VMEM((BPW // ICH, ICH), jnp.int32),
            pltpu.VMEM((BPW // ICH, ICH), jnp.int32),
            pltpu.VMEM((2, BPW // ICH, ICH), jnp.int32),
            pltpu.VMEM((BPW, D), jnp.float32),
            pltpu.VMEM((BPW, D), jnp.float32),
            pltpu.VMEM((BPW,), jnp.float32),
            pltpu.SemaphoreType.DMA,
        ],
    )
    def k2b(invm_hbm, invt_hbm, mv_hbm, tv_hbm, out_hbm,
            im, it, pp, mrows, trows, out_v, sem):
        wid = lax.axis_index("s") * 2 + lax.axis_index("c")
        base = wid * BPW

        def merge(inv_hbm, dst):
            for h in range(2):
                for j in range(BPW // ICH):
                    pltpu.sync_copy(
                        inv_hbm.at[h, pl.ds(base + j * ICH, ICH)],
                        pp.at[h, j])
            for j in range(BPW // ICH):
                for g in range(ICH // L):
                    off = pl.ds(g * L, L)
                    v = pp[0, j, off] + pp[1, j, off] - 1
                    dst[j, off] = jnp.maximum(v, 0)

        merge(invm_hbm, im)
        merge(invt_hbm, it)
        copies = []
        for j in range(BPW // ICH):
            copies.append(pltpu.async_copy(
                mv_hbm.at[im.at[j]], mrows.at[pl.ds(j * ICH, ICH)], sem))
            copies.append(pltpu.async_copy(
                tv_hbm.at[it.at[j]], trows.at[pl.ds(j * ICH, ICH)], sem))
        for cpy in copies:
            cpy.wait()

        iota = lax.iota(jnp.int32, L)

        def group_body(g, carry):
            row = pl.multiple_of(g * L, L) + iota
            acc = jnp.zeros((L,), jnp.float32)
            for d in range(D):
                col = jnp.full((L,), d, jnp.int32)
                mv = plsc.load_gather(mrows, [row, col])
                tv = plsc.load_gather(trows, [row, col])
                acc = acc + mv * tv
            out_v[pl.ds(pl.multiple_of(g * L, L), L)] = acc
            return carry

        lax.fori_loop(0, BPW // L, group_body, 0)
        pltpu.sync_copy(out_v, out_hbm.at[pl.ds(base, BPW)])

    return k2b


_k1 = _make_k1()
_k2a = _make_k2a()
_k2b = _make_k2b()


def kernel(model, task, model_factors, task_factors):
    model = model.astype(jnp.int32)
    task = task.astype(jnp.int32)
    mrows, trows, pmb, ptb = _k1(
        model, task, model_factors.T, task_factors.T)
    inv_m, inv_t = _k2a(pmb, ptb)
    return _k2b(inv_m, inv_t,
                mrows.reshape(NREG, D), trows.reshape(NREG, D))


# speedup vs baseline: 1.9524x; 1.9524x over previous
import functools

import jax
import jax.numpy as jnp
from jax import lax
from jax.experimental import pallas as pl
from jax.experimental.pallas import tpu as pltpu
from jax.experimental.pallas import tpu_sc as plsc

D = 32
B = 16384
L = 16
NW = 32
BPW = B // NW
KCH = 8
NKCH = BPW // KCH
ICH = 128


def _make_k1():
    mesh = plsc.VectorSubcoreMesh(core_axis_name="c", subcore_axis_name="s")

    @functools.partial(
        pl.kernel,
        mesh=mesh,
        out_type=jax.ShapeDtypeStruct((B * D,), jnp.float32),
        compiler_params=pltpu.CompilerParams(
            needs_layout_passes=False, use_tc_tiling_on_sc=True),
        scratch_types=[
            pltpu.VMEM((BPW,), jnp.int32),
            pltpu.VMEM((2, KCH, D, 128), jnp.float32),
            pltpu.VMEM((BPW * D,), jnp.float32),
            pltpu.SemaphoreType.DMA,
            pltpu.SemaphoreType.DMA,
        ],
    )
    def k1(model_hbm, mt_hbm, mout_hbm, idx_m, cols, rows, sem0, sem1):
        wid = lax.axis_index("s") * 2 + lax.axis_index("c")
        base = wid * BPW

        pltpu.sync_copy(model_hbm.at[pl.ds(base, BPW)], idx_m)
        iota = lax.iota(jnp.int32, L)
        sems = (sem0, sem1)

        def load_idx(p):
            return idx_m[pl.ds(pl.multiple_of(p * 2 * KCH, 2 * KCH), L)]

        def fire(col_base, parity):
            for i in range(KCH):
                cb = pl.multiple_of(col_base[parity * KCH + i], 128)
                pltpu.async_copy(
                    mt_hbm.at[:, pl.ds(cb, 128)],
                    cols.at[parity, i], sems[parity])

        def drain_extract(p, lane, parity):
            for i in range(KCH):
                pltpu.make_async_copy(
                    mt_hbm.at[:, pl.ds(0, 128)],
                    cols.at[parity, i], sems[parity]).wait()
            c0 = pl.multiple_of((p * 2 + parity) * KCH, KCH)
            for i in range(KCH):
                lane_col = jnp.full((L,), lane[parity * KCH + i], jnp.int32)
                v0 = plsc.load_gather(cols.at[parity, i], [iota, lane_col])
                v1 = plsc.load_gather(cols.at[parity, i], [iota + L, lane_col])
                r0 = pl.multiple_of((c0 + i) * D, D)
                rows[pl.ds(r0, L)] = v0
                rows[pl.ds(r0 + L, L)] = v1

        idx0 = load_idx(0)
        fire((idx0 >> 7) << 7, 0)

        def pair_body(p, carry):
            idx_vec = load_idx(p)
            col_base = (idx_vec >> 7) << 7
            lane = idx_vec & 127
            fire(col_base, 1)
            drain_extract(p, lane, 0)

            @pl.when(p + 1 < NKCH // 2)
            def _():
                idx_nxt = load_idx(p + 1)
                fire((idx_nxt >> 7) << 7, 0)

            drain_extract(p, lane, 1)
            return carry

        lax.fori_loop(0, NKCH // 2, pair_body, 0)
        pltpu.sync_copy(rows, mout_hbm.at[pl.ds(base * D, BPW * D)])

    return k1


def _make_k2():
    mesh = plsc.VectorSubcoreMesh(core_axis_name="c", subcore_axis_name="s")

    @functools.partial(
        pl.kernel,
        mesh=mesh,
        out_type=jax.ShapeDtypeStruct((B,), jnp.float32),
        compiler_params=pltpu.CompilerParams(
            needs_layout_passes=False, use_tc_tiling_on_sc=False),
        scratch_types=[
            pltpu.VMEM((BPW // ICH, ICH), jnp.int32),
            pltpu.VMEM((BPW, D), jnp.float32),
            pltpu.VMEM((BPW * D,), jnp.float32),
            pltpu.VMEM((BPW,), jnp.float32),
            pltpu.SemaphoreType.DMA,
        ],
    )
    def k2(task_hbm, tf_hbm, mvec_hbm, out_hbm,
           idx_t, trows, mrows, out_v, sem):
        wid = lax.axis_index("s") * 2 + lax.axis_index("c")
        base = wid * BPW

        for j in range(BPW // ICH):
            pltpu.sync_copy(task_hbm.at[pl.ds(base + j * ICH, ICH)],
                            idx_t.at[j])
        copies = [pltpu.async_copy(
            mvec_hbm.at[pl.ds(base * D, BPW * D)], mrows, sem)]
        for j in range(BPW // ICH):
            copies.append(pltpu.async_copy(
                tf_hbm.at[idx_t.at[j]],
                trows.at[pl.ds(j * ICH, ICH)], sem))
        for cpy in copies:
            cpy.wait()

        iota = lax.iota(jnp.int32, L)

        def group_body(g, carry):
            row = pl.multiple_of(g * L, L) + iota
            flat = row * D
            acc = jnp.zeros((L,), jnp.float32)
            for d in range(D):
                col = jnp.full((L,), d, jnp.int32)
                mv = plsc.load_gather(mrows, [flat + d])
                tv = plsc.load_gather(trows, [row, col])
                acc = acc + mv * tv
            out_v[pl.ds(pl.multiple_of(g * L, L), L)] = acc
            return carry

        lax.fori_loop(0, BPW // L, group_body, 0)
        pltpu.sync_copy(out_v, out_hbm.at[pl.ds(base, BPW)])

    return k2


_k1 = _make_k1()
_k2 = _make_k2()


def kernel(model, task, model_factors, task_factors):
    model = model.astype(jnp.int32)
    task = task.astype(jnp.int32)
    mvec = _k1(model, model_factors.T)
    return _k2(task, task_factors, mvec)
